# Initial kernel scaffold; baseline (speedup 1.0000x reference)
import jax
import jax.numpy as jnp
from jax.experimental import pallas as pl


def kernel(x, edge_index, W_in, b_in, W_g1, b_g1, gamma1, beta1, W_g2, b_g2, gamma2, beta2, W_gat, att_src, att_dst, b_gat, gamma_g, beta_g, W_h1, b_h1, W_h2, b_h2):
    raise NotImplementedError("write your pallas kernel here")


# speedup vs baseline: 19.6483x; 19.6483x over previous
import functools

import jax
import jax.numpy as jnp
from jax import lax
from jax.experimental import pallas as pl
from jax.experimental.pallas import tpu as pltpu
from jax.experimental.pallas import tpu_sc as plsc

N = 10000
NP = 10240
D = 256
DH = 128
E = 160000
HEADS = 4

NC = 2
NS = 16
L = 16
SR = NP // NS
EPT = E // NS
CH_E = 80
NB = EPT // CH_E
FN = float(N)
SEG = 25
NSEG = NB // SEG

_mesh = plsc.VectorSubcoreMesh(core_axis_name="c", subcore_axis_name="s",
                               num_cores=NC, num_subcores=NS)


@functools.partial(
    pl.kernel,
    out_type=jax.ShapeDtypeStruct((NP,), jnp.float32),
    mesh=_mesh,
    scratch_types=[
        pltpu.VMEM((SEG, CH_E), jnp.int32),
        pltpu.VMEM((SR,), jnp.float32),
        pltpu.VMEM_SHARED((NP,), jnp.float32),
    ],
)
def _sc_deg(dstT_hbm, ones_hbm, deg_hbm, dst_slab, onesv, deg_s):
    c = lax.axis_index("c")
    t = lax.axis_index("s")
    tbase = t * SR
    pltpu.sync_copy(ones_hbm, onesv)
    pltpu.sync_copy(onesv, deg_s.at[pl.ds(tbase, SR)])
    plsc.subcore_barrier()

    def _seg(seg, carry):
        pltpu.sync_copy(dstT_hbm.at[t, seg], dst_slab)

        def _b(b, cc):
            pltpu.sync_copy(onesv.at[pl.ds(0, CH_E)],
                            deg_s.at[dst_slab.at[b]], add=True)
            return cc

        lax.fori_loop(0, SEG, _b, 0)
        return carry

    lax.fori_loop(0, NSEG, _seg, 0)
    plsc.subcore_barrier()

    @pl.when(c == 0)
    def _():
        pltpu.sync_copy(deg_s.at[pl.ds(tbase, SR)], onesv)
        pltpu.sync_copy(onesv, deg_hbm.at[pl.ds(tbase, SR)])


@functools.partial(
    pl.kernel,
    out_type=jax.ShapeDtypeStruct((2 * NP, DH), jnp.float32),
    mesh=_mesh,
    scratch_types=[
        pltpu.VMEM((SEG, CH_E), jnp.int32),
        pltpu.VMEM((SEG, CH_E), jnp.int32),
        pltpu.VMEM((CH_E, DH), jnp.float32),
        pltpu.VMEM((CH_E, DH), jnp.float32),
        pltpu.SemaphoreType.DMA,
        pltpu.SemaphoreType.DMA,
        pltpu.VMEM_SHARED((NP, DH), jnp.float32),
    ],
)
def _sc_gcn(hw2_hbm, srcT_hbm, dstT_hbm, out_hbm,
            sseg, dseg, rows0, rows1, sem0, sem1, acc_s):
    c = lax.axis_index("c")
    t = lax.axis_index("s")
    cNP = c * NP
    tbase = t * SR

    for k in range(SR // CH_E):
        pltpu.sync_copy(hw2_hbm.at[pl.ds(cNP + tbase + k * CH_E, CH_E)], rows0)
        pltpu.sync_copy(rows0, acc_s.at[pl.ds(tbase + k * CH_E, CH_E)])
    plsc.subcore_barrier()

    def _fire(b, buf, sem):
        return pltpu.async_copy(hw2_hbm.at[sseg.at[b]], buf, sem)

    def _seg_body(seg, carry):
        pltpu.sync_copy(srcT_hbm.at[t, seg], sseg)
        pltpu.sync_copy(dstT_hbm.at[t, seg], dseg)

        def _adj(b, cc):
            for j in range(CH_E // L):
                sl = pl.ds(j * L, L)
                sseg[b, sl] = sseg[b, sl] + cNP
            return cc

        lax.fori_loop(0, SEG, _adj, 0)

        _fire(0, rows0, sem0)
        _fire(1, rows1, sem1)

        def _pair(j, cc):
            b0 = 2 * j
            pltpu.make_async_copy(hw2_hbm.at[sseg.at[b0]], rows0, sem0).wait()
            pltpu.sync_copy(rows0, acc_s.at[dseg.at[b0]], add=True)

            @pl.when(b0 + 2 < SEG)
            def _():
                _fire(b0 + 2, rows0, sem0)

            b1 = b0 + 1
            pltpu.make_async_copy(hw2_hbm.at[sseg.at[b1]], rows1, sem1).wait()
            pltpu.sync_copy(rows1, acc_s.at[dseg.at[b1]], add=True)

            @pl.when(b1 + 2 < SEG)
            def _():
                _fire(b1 + 2, rows1, sem1)

            return cc

        lax.fori_loop(0, SEG // 2, _pair, 0)
        pltpu.make_async_copy(hw2_hbm.at[sseg.at[SEG - 1]], rows0, sem0).wait()
        pltpu.sync_copy(rows0, acc_s.at[dseg.at[SEG - 1]], add=True)
        return carry

    lax.fori_loop(0, NSEG, _seg_body, 0)

    plsc.subcore_barrier()
    for k in range(SR // CH_E):
        pltpu.sync_copy(acc_s.at[pl.ds(tbase + k * CH_E, CH_E)], rows0)
        pltpu.sync_copy(rows0, out_hbm.at[pl.ds(cNP + tbase + k * CH_E, CH_E)])


@functools.partial(
    pl.kernel,
    out_type=(jax.ShapeDtypeStruct((2 * NP, DH), jnp.float32),
              jax.ShapeDtypeStruct((NC, NS, NSEG, SEG, 2, CH_E), jnp.float32),
              jax.ShapeDtypeStruct((4 * NP,), jnp.float32)),
    mesh=_mesh,
    scratch_types=[
        pltpu.VMEM((SEG, CH_E), jnp.int32),
        pltpu.VMEM((SEG, CH_E), jnp.int32),
        pltpu.VMEM((CH_E,), jnp.int32),
        pltpu.VMEM((2, CH_E), jnp.float32),
        pltpu.VMEM((CH_E,), jnp.float32),
        pltpu.VMEM((CH_E,), jnp.float32),
        pltpu.VMEM((CH_E,), jnp.float32),
        pltpu.VMEM((SR,), jnp.float32),
        pltpu.VMEM((SR,), jnp.float32),
        pltpu.VMEM((SR,), jnp.float32),
        pltpu.VMEM((SR,), jnp.float32),
        pltpu.VMEM((2, 128), jnp.float32),
        pltpu.VMEM((CH_E, DH), jnp.float32),
        pltpu.VMEM((CH_E, DH), jnp.float32),
        pltpu.SemaphoreType.DMA,
        pltpu.VMEM_SHARED((NP,), jnp.float32),
        pltpu.VMEM_SHARED((NP,), jnp.float32),
        pltpu.VMEM_SHARED((NP, DH), jnp.float32),
    ],
)
def _sc_gat(xl2_hbm, al_hbm, ar_hbm, m2_hbm, srcT_hbm, dstT_hbm,
            out_hbm, ee_hbm, dn_hbm,
            sseg, dseg, adjb, ebuf, arb, dnb0, dnb1,
            stA, stB, esA, esB, mv, rows, obuf, sem0,
            dn_s0, dn_s1, acc_s):
    c = lax.axis_index("c")
    t = lax.axis_index("s")
    cNP = c * NP
    c2NP = c * (2 * NP)
    tbase = t * SR

    pltpu.sync_copy(m2_hbm, mv)
    zmax = mv[0, pl.ds(0, L)] + mv[1, pl.ds(0, L)]
    m = jnp.where(zmax >= 0.0, zmax, 0.2 * zmax)

    def _leaky_ee(a, b):
        z = a + b
        z = jnp.where(z >= 0.0, z, 0.2 * z)
        return jnp.exp(z - m)

    for h, es in ((0, esA), (1, esB)):
        off = c2NP + h * NP + tbase
        pltpu.sync_copy(al_hbm.at[pl.ds(off, SR)], es)
        pltpu.sync_copy(ar_hbm.at[pl.ds(off, SR)], stA)
        for j in range(SR // L):
            sl = pl.ds(j * L, L)
            es[sl] = _leaky_ee(es[sl], stA[sl])
    pltpu.sync_copy(esA, dn_s0.at[pl.ds(tbase, SR)])
    pltpu.sync_copy(esB, dn_s1.at[pl.ds(tbase, SR)])
    plsc.subcore_barrier()

    def _mkadj(slab, b, off):
        for j in range(CH_E // L):
            sl = pl.ds(j * L, L)
            adjb[sl] = slab[b, sl] + off

    def _load_seg(seg):
        pltpu.sync_copy(srcT_hbm.at[t, seg], sseg)
        pltpu.sync_copy(dstT_hbm.at[t, seg], dseg)

    def _att_seg(seg, carry):
        _load_seg(seg)

        def _att_batch(b, cc):
            for h in range(2):
                _mkadj(sseg, b, c2NP + h * NP)
                pltpu.sync_copy(al_hbm.at[adjb], ebuf.at[h])
                _mkadj(dseg, b, c2NP + h * NP)
                pltpu.sync_copy(ar_hbm.at[adjb], arb)
                for j in range(CH_E // L):
                    sl = pl.ds(j * L, L)
                    ebuf[h, sl] = _leaky_ee(ebuf[h, sl], arb[sl])
            pltpu.sync_copy(ebuf.at[0], dn_s0.at[dseg.at[b]], add=True)
            pltpu.sync_copy(ebuf.at[1], dn_s1.at[dseg.at[b]], add=True)
            pltpu.sync_copy(ebuf, ee_hbm.at[c, t, seg, b])
            return cc

        lax.fori_loop(0, SEG, _att_batch, 0)
        return carry

    lax.fori_loop(0, NSEG, _att_seg, 0)
    plsc.subcore_barrier()

    pltpu.sync_copy(dn_s0.at[pl.ds(tbase, SR)], stA)
    pltpu.sync_copy(dn_s1.at[pl.ds(tbase, SR)], stB)
    pltpu.sync_copy(stA, dn_hbm.at[pl.ds(c2NP + tbase, SR)])
    pltpu.sync_copy(stB, dn_hbm.at[pl.ds(c2NP + NP + tbase, SR)])

    def _scale_chunk(a0v, a1v, j):
        for rl in range(L):
            r = j * L + rl
            s0 = jnp.full((L,), a0v[rl], jnp.float32)
            s1 = jnp.full((L,), a1v[rl], jnp.float32)
            for q in range(64 // L):
                sl = pl.ds(q * L, L)
                obuf[r, sl] = rows[r, sl] * s0
            for q in range(64 // L):
                sl = pl.ds(64 + q * L, L)
                obuf[r, sl] = rows[r, sl] * s1

    for k in range(SR // CH_E):
        pltpu.sync_copy(xl2_hbm.at[pl.ds(cNP + tbase + k * CH_E, CH_E)], rows)
        for j in range(CH_E // L):
            q0 = k * CH_E + j * L
            sl = pl.ds(q0, L)
            a0v = esA[sl] / (stA[sl] + 1e-16)
            a1v = esB[sl] / (stB[sl] + 1e-16)
            _scale_chunk(a0v, a1v, j)
        pltpu.sync_copy(obuf, acc_s.at[pl.ds(tbase + k * CH_E, CH_E)])
    plsc.subcore_barrier()

    def _msg_seg(seg, carry):
        _load_seg(seg)

        def _edge_batch(b, cc):
            pltpu.sync_copy(ee_hbm.at[c, t, seg, b], ebuf)
            _mkadj(dseg, b, c2NP)
            pltpu.sync_copy(dn_hbm.at[adjb], dnb0)
            _mkadj(dseg, b, c2NP + NP)
            pltpu.sync_copy(dn_hbm.at[adjb], dnb1)
            _mkadj(sseg, b, cNP)
            pltpu.async_copy(xl2_hbm.at[adjb], rows, sem0).wait()
            for j in range(CH_E // L):
                sl = pl.ds(j * L, L)
                a0v = ebuf[0, sl] / (dnb0[sl] + 1e-16)
                a1v = ebuf[1, sl] / (dnb1[sl] + 1e-16)
                _scale_chunk(a0v, a1v, j)
            pltpu.sync_copy(obuf, acc_s.at[dseg.at[b]], add=True)
            return cc

        lax.fori_loop(0, SEG, _edge_batch, 0)
        return carry

    lax.fori_loop(0, NSEG, _msg_seg, 0)
    plsc.subcore_barrier()

    for k in range(SR // CH_E):
        pltpu.sync_copy(acc_s.at[pl.ds(tbase + k * CH_E, CH_E)], rows)
        pltpu.sync_copy(rows, out_hbm.at[pl.ds(cNP + tbase + k * CH_E, CH_E)])


BR = 640
GRID = NP // BR


def _dot(a, b):
    return jnp.dot(a, b, preferred_element_type=jnp.float32)


def _row_spec(cols):
    return pl.BlockSpec((BR, cols), lambda i: (i, 0))


def _half_spec():
    return pl.BlockSpec((2, BR, DH), lambda i: (0, i, 0))


def _full_spec(shape):
    nd = len(shape)
    return pl.BlockSpec(shape, lambda i: (0,) * nd)


def _tc_mm1_body(x_ref, wi_ref, bi_ref, wg_ref, deg_ref, h0_ref, hw_ref):
    h0 = jnp.maximum(_dot(x_ref[...], wi_ref[...]) + bi_ref[...], 0.0)
    dinv = lax.rsqrt(deg_ref[...])
    z = _dot(h0, wg_ref[...]) * dinv
    h0_ref[...] = h0
    hw_ref[0] = z[:, :DH]
    hw_ref[1] = z[:, DH:]


def _tc_mm1(x_p, W_in, b_in, W_g1, deg):
    return pl.pallas_call(
        _tc_mm1_body,
        grid=(GRID,),
        in_specs=[_row_spec(D), _full_spec((D, D)), _full_spec((1, D)),
                  _full_spec((D, D)), _row_spec(1)],
        out_specs=(_row_spec(D), _half_spec()),
        out_shape=(jax.ShapeDtypeStruct((NP, D), jnp.float32),
                   jax.ShapeDtypeStruct((2, NP, DH), jnp.float32)),
    )(x_p, W_in, b_in, W_g1, deg)


def _tc_stat_body(acc_ref, deg_ref, b_ref, o_ref):
    i = pl.program_id(0)
    y = jnp.concatenate([acc_ref[0], acc_ref[1]], axis=1)
    y = y * lax.rsqrt(deg_ref[...]) + b_ref[...]
    rid = lax.broadcasted_iota(jnp.int32, (BR, 1), 0) + i * BR
    mask = rid < N
    y = jnp.where(mask, y, 0.0)
    s1 = jnp.sum(y, axis=0, keepdims=True)
    s2 = jnp.sum(y * y, axis=0, keepdims=True)
    blk = jnp.concatenate([s1, s2], axis=0)

    @pl.when(i == 0)
    def _():
        o_ref[...] = blk

    @pl.when(i > 0)
    def _():
        o_ref[...] = o_ref[...] + blk


def _tc_stat(acc, deg, b):
    return pl.pallas_call(
        _tc_stat_body,
        grid=(GRID,),
        in_specs=[_half_spec(), _row_spec(1), _full_spec((1, D))],
        out_specs=_full_spec((2, D)),
        out_shape=jax.ShapeDtypeStruct((2, D), jnp.float32),
    )(acc, deg, b)


def _bn_from_stats(y, st_ref, gamma_ref, beta_ref):
    mu = st_ref[0:1, :] / FN
    var = st_ref[1:2, :] / FN - mu * mu
    return (y - mu) * lax.rsqrt(var + 1e-5) * gamma_ref[...] + beta_ref[...]


def _tc_fuse1_body(h_ref, acc_ref, deg_ref, b_ref, g_ref, be_ref, st_ref,
                   w_ref, h1_ref, hw_ref):
    dinv = lax.rsqrt(deg_ref[...])
    y = jnp.concatenate([acc_ref[0], acc_ref[1]], axis=1) * dinv + b_ref[...]
    h1 = h_ref[...] + jnp.maximum(_bn_from_stats(y, st_ref, g_ref, be_ref), 0.0)
    z = _dot(h1, w_ref[...]) * dinv
    h1_ref[...] = h1
    hw_ref[0] = z[:, :DH]
    hw_ref[1] = z[:, DH:]


def _tc_fuse1(h, acc, deg, b, gamma, beta, st, W_next):
    return pl.pallas_call(
        _tc_fuse1_body,
        grid=(GRID,),
        in_specs=[_row_spec(D), _half_spec(), _row_spec(1),
                  _full_spec((1, D)), _full_spec((1, D)), _full_spec((1, D)),
                  _full_spec((2, D)), _full_spec((D, D))],
        out_specs=(_row_spec(D), _half_spec()),
        out_shape=(jax.ShapeDtypeStruct((NP, D), jnp.float32),
                   jax.ShapeDtypeStruct((2, NP, DH), jnp.float32)),
    )(h, acc, deg, b, gamma, beta, st, W_next)


def _tc_fuse2_body(h_ref, acc_ref, deg_ref, b_ref, g_ref, be_ref, st_ref,
                   wg_ref, asrc_ref, adst_ref,
                   h2_ref, xl_ref, al_ref, ar_ref, m_ref):
    i = pl.program_id(0)
    dinv = lax.rsqrt(deg_ref[...])
    y = jnp.concatenate([acc_ref[0], acc_ref[1]], axis=1) * dinv + b_ref[...]
    h2 = h_ref[...] + jnp.maximum(_bn_from_stats(y, st_ref, g_ref, be_ref), 0.0)
    xl = _dot(h2, wg_ref[...])
    asrc = asrc_ref[...]
    adst = adst_ref[...]
    al_cols = []
    ar_cols = []
    for h in range(HEADS):
        xh = xl[:, 64 * h:64 * (h + 1)]
        al_cols.append(jnp.sum(xh * asrc[h:h + 1, :], axis=1, keepdims=True))
        ar_cols.append(jnp.sum(xh * adst[h:h + 1, :], axis=1, keepdims=True))
    al = jnp.concatenate(al_cols, axis=1)
    ar = jnp.concatenate(ar_cols, axis=1)
    h2_ref[...] = h2
    xl_ref[0] = xl[:, :DH]
    xl_ref[1] = xl[:, DH:]
    for ci in range(2):
        for h in range(2):
            al_ref[ci, h, :] = al[:, 2 * ci + h]
            ar_ref[ci, h, :] = ar[:, 2 * ci + h]
    bal = jnp.full((1, 128), jnp.max(al), jnp.float32)
    bar = jnp.full((1, 128), jnp.max(ar), jnp.float32)
    blk = jnp.concatenate([bal, bar], axis=0)

    @pl.when(i == 0)
    def _():
        m_ref[...] = blk

    @pl.when(i > 0)
    def _():
        m_ref[...] = jnp.maximum(m_ref[...], blk)


def _tc_fuse2(h, acc, deg, b, gamma, beta, st, W_gat, att_src, att_dst):
    return pl.pallas_call(
        _tc_fuse2_body,
        grid=(GRID,),
        in_specs=[_row_spec(D), _half_spec(), _row_spec(1),
                  _full_spec((1, D)), _full_spec((1, D)), _full_spec((1, D)),
                  _full_spec((2, D)), _full_spec((D, D)),
                  _full_spec((HEADS, 64)), _full_spec((HEADS, 64))],
        out_specs=(_row_spec(D), _half_spec(),
                   pl.BlockSpec((2, 2, BR), lambda i: (0, 0, i)),
                   pl.BlockSpec((2, 2, BR), lambda i: (0, 0, i)),
                   _full_spec((2, 128))),
        out_shape=(jax.ShapeDtypeStruct((NP, D), jnp.float32),
                   jax.ShapeDtypeStruct((2, NP, DH), jnp.float32),
                   jax.ShapeDtypeStruct((2, 2, NP), jnp.float32),
                   jax.ShapeDtypeStruct((2, 2, NP), jnp.float32),
                   jax.ShapeDtypeStruct((2, 128), jnp.float32)),
    )(h, acc, deg, b, gamma, beta, st, W_gat, att_src, att_dst)


def _tc_head_body(hg_ref, h2_ref, b_ref, g_ref, be_ref, st_ref,
                  w1_ref, b1_ref, w2_ref, b2_ref, o_ref):
    y = jnp.concatenate([hg_ref[0], hg_ref[1]], axis=1) + b_ref[...]
    h3 = _bn_from_stats(y, st_ref, g_ref, be_ref) + h2_ref[...]
    tmid = jnp.maximum(_dot(h3, w1_ref[...]) + b1_ref[...], 0.0)
    o_ref[...] = _dot(tmid, w2_ref[...]) + b2_ref[...]


def _tc_head(hg, h2, b_gat, gamma_g, beta_g, st, W_h1, b_h1, W_h2, b_h2):
    return pl.pallas_call(
        _tc_head_body,
        grid=(GRID,),
        in_specs=[_half_spec(), _row_spec(D), _full_spec((1, D)),
                  _full_spec((1, D)), _full_spec((1, D)), _full_spec((2, D)),
                  _full_spec((D, 128)), _full_spec((1, 128)),
                  _full_spec((128, 1)), _full_spec((1, 1))],
        out_specs=_row_spec(1),
        out_shape=jax.ShapeDtypeStruct((NP, 1), jnp.float32),
    )(hg, h2, b_gat, gamma_g, beta_g, st, W_h1, b_h1, W_h2, b_h2)


_DBG_DEG = False
_DBG_GCN = False
_DBG_GAT = False


def _jnp_deg(dst_t):
    d = dst_t.reshape(-1)
    deg = jax.ops.segment_sum(jnp.ones((E,), jnp.float32), d, num_segments=NP)
    return deg + 1.0


def _jnp_gcn(hw2f, src_t, dst_t):
    hw = hw2f.reshape(2, NP, DH)
    s = src_t.reshape(-1)
    d = dst_t.reshape(-1)
    agg = hw + jax.vmap(
        lambda h: jax.ops.segment_sum(h[s], d, num_segments=NP))(hw)
    return agg.reshape(2 * NP, DH)


def _jnp_gat(xl2f, al_f, ar_f, m2, src_t, dst_t):
    xl = xl2f.reshape(2, NP, DH)
    al = al_f.reshape(2, 2, NP)
    ar = ar_f.reshape(2, 2, NP)
    s = src_t.reshape(-1)
    d = dst_t.reshape(-1)
    zm = m2[0, 0] + m2[1, 0]
    m = jnp.where(zm >= 0, zm, 0.2 * zm)
    outs = []
    for c in range(2):
        halves = []
        for h in range(2):
            z = al[c, h][s] + ar[c, h][d]
            z = jnp.where(z >= 0, z, 0.2 * z)
            ee = jnp.exp(z - m)
            zs = al[c, h] + ar[c, h]
            zs = jnp.where(zs >= 0, zs, 0.2 * zs)
            ees = jnp.exp(zs - m)
            dn = jax.ops.segment_sum(ee, d, num_segments=NP) + ees
            xh = xl[c][:, 64 * h:64 * (h + 1)]
            msg = jax.ops.segment_sum(xh[s] * (ee / (dn[d] + 1e-16))[:, None],
                                      d, num_segments=NP)
            msg = msg + xh * (ees / (dn + 1e-16))[:, None]
            halves.append(msg)
        outs.append(jnp.concatenate(halves, axis=1))
    return jnp.stack(outs, 0).reshape(2 * NP, DH)


@jax.jit
def kernel(x, edge_index, W_in, b_in, W_g1, b_g1, gamma1, beta1,
           W_g2, b_g2, gamma2, beta2, W_gat, att_src, att_dst, b_gat,
           gamma_g, beta_g, W_h1, b_h1, W_h2, b_h2):
    x_p = jnp.pad(x, ((0, NP - N), (0, 0)))
    src_t = edge_index[0].reshape(NS, NSEG, SEG, CH_E)
    dst_t = edge_index[1].reshape(NS, NSEG, SEG, CH_E)
    ones_sr = jnp.ones((SR,), jnp.float32)

    r2 = lambda v: v[None, :]
    b_in2, b_g1_2, b_g2_2, b_gat2 = r2(b_in), r2(b_g1), r2(b_g2), r2(b_gat)
    gamma1_2, beta1_2 = r2(gamma1), r2(beta1)
    gamma2_2, beta2_2 = r2(gamma2), r2(beta2)
    gamma_g2, beta_g2 = r2(gamma_g), r2(beta_g)
    b_h1_2 = r2(b_h1)
    b_h2_2 = r2(b_h2)

    deg = _jnp_deg(dst_t) if _DBG_DEG else _sc_deg(dst_t, ones_sr)
    deg2 = deg[:, None]

    h0, hw1 = _tc_mm1(x_p, W_in, b_in2, W_g1, deg2)
    agg1 = (_jnp_gcn(hw1.reshape(2 * NP, DH), src_t, dst_t) if _DBG_GCN
            else _sc_gcn(hw1.reshape(2 * NP, DH), src_t, dst_t)).reshape(2, NP, DH)
    st1 = _tc_stat(agg1, deg2, b_g1_2)
    h1, hw2 = _tc_fuse1(h0, agg1, deg2, b_g1_2, gamma1_2, beta1_2, st1, W_g2)
    agg2 = (_jnp_gcn(hw2.reshape(2 * NP, DH), src_t, dst_t) if _DBG_GCN
            else _sc_gcn(hw2.reshape(2 * NP, DH), src_t, dst_t)).reshape(2, NP, DH)
    st2 = _tc_stat(agg2, deg2, b_g2_2)
    h2, xl2, al_ph, ar_ph, m2 = _tc_fuse2(
        h1, agg2, deg2, b_g2_2, gamma2_2, beta2_2, st2,
        W_gat, att_src, att_dst)
    if _DBG_GAT:
        hg = _jnp_gat(xl2.reshape(2 * NP, DH), al_ph.reshape(4 * NP),
                      ar_ph.reshape(4 * NP), m2, src_t, dst_t).reshape(2, NP, DH)
    else:
        hg = _sc_gat(xl2.reshape(2 * NP, DH), al_ph.reshape(4 * NP),
                     ar_ph.reshape(4 * NP), m2, src_t, dst_t)[0].reshape(2, NP, DH)
    ones_np = jnp.ones((NP, 1), jnp.float32)
    st3 = _tc_stat(hg, ones_np, b_gat2)
    out = _tc_head(hg, h2, b_gat2, gamma_g2, beta_g2, st3,
                   W_h1, b_h1_2, W_h2, b_h2_2)
    return out[:N]


# speedup vs baseline: 30.0447x; 1.5291x over previous
import functools

import jax
import jax.numpy as jnp
from jax import lax
from jax.experimental import pallas as pl
from jax.experimental.pallas import tpu as pltpu
from jax.experimental.pallas import tpu_sc as plsc

N = 10000
NP = 10240
D = 256
DH = 128
E = 160000
HEADS = 4

NC = 2
NS = 16
L = 16
SR = NP // NS
EPT = E // NS
CH_E = 80
NB = EPT // CH_E
FN = float(N)
SEG = 25
NSEG = NB // SEG

_mesh = plsc.VectorSubcoreMesh(core_axis_name="c", subcore_axis_name="s",
                               num_cores=NC, num_subcores=NS)


@functools.partial(
    pl.kernel,
    out_type=jax.ShapeDtypeStruct((NP,), jnp.float32),
    mesh=_mesh,
    scratch_types=[
        pltpu.VMEM((SEG, CH_E), jnp.int32),
        pltpu.VMEM((SR,), jnp.float32),
        pltpu.VMEM_SHARED((NP,), jnp.float32),
    ],
)
def _sc_deg(dstT_hbm, ones_hbm, deg_hbm, dst_slab, onesv, deg_s):
    c = lax.axis_index("c")
    t = lax.axis_index("s")
    tbase = t * SR
    pltpu.sync_copy(ones_hbm, onesv)
    pltpu.sync_copy(onesv, deg_s.at[pl.ds(tbase, SR)])
    plsc.subcore_barrier()

    def _seg(seg, carry):
        pltpu.sync_copy(dstT_hbm.at[t, seg], dst_slab)

        def _b(b, cc):
            pltpu.sync_copy(onesv.at[pl.ds(0, CH_E)],
                            deg_s.at[dst_slab.at[b]], add=True)
            return cc

        lax.fori_loop(0, SEG, _b, 0)
        return carry

    lax.fori_loop(0, NSEG, _seg, 0)
    plsc.subcore_barrier()

    @pl.when(c == 0)
    def _():
        pltpu.sync_copy(deg_s.at[pl.ds(tbase, SR)], onesv)
        pltpu.sync_copy(onesv, deg_hbm.at[pl.ds(tbase, SR)])


@functools.partial(
    pl.kernel,
    out_type=jax.ShapeDtypeStruct((2 * NP, DH), jnp.float32),
    mesh=_mesh,
    scratch_types=[
        pltpu.VMEM((SEG, CH_E), jnp.int32),
        pltpu.VMEM((SEG, CH_E), jnp.int32),
        pltpu.VMEM((CH_E, DH), jnp.float32),
        pltpu.VMEM((CH_E, DH), jnp.float32),
        pltpu.SemaphoreType.DMA,
        pltpu.SemaphoreType.DMA,
        pltpu.VMEM_SHARED((NP, DH), jnp.float32),
    ],
)
def _sc_gcn(hw2_hbm, srcT_hbm, dstT_hbm, out_hbm,
            sseg, dseg, rows0, rows1, sem0, sem1, acc_s):
    c = lax.axis_index("c")
    t = lax.axis_index("s")
    cNP = c * NP
    tbase = t * SR

    for k in range(SR // CH_E):
        pltpu.sync_copy(hw2_hbm.at[pl.ds(cNP + tbase + k * CH_E, CH_E)], rows0)
        pltpu.sync_copy(rows0, acc_s.at[pl.ds(tbase + k * CH_E, CH_E)])
    plsc.subcore_barrier()

    def _fire(b, buf, sem):
        return pltpu.async_copy(hw2_hbm.at[sseg.at[b]], buf, sem)

    def _seg_body(seg, carry):
        pltpu.sync_copy(srcT_hbm.at[t, seg], sseg)
        pltpu.sync_copy(dstT_hbm.at[t, seg], dseg)

        def _adj(b, cc):
            for j in range(CH_E // L):
                sl = pl.ds(j * L, L)
                sseg[b, sl] = sseg[b, sl] + cNP
            return cc

        lax.fori_loop(0, SEG, _adj, 0)

        _fire(0, rows0, sem0)
        _fire(1, rows1, sem1)

        def _pair(j, cc):
            b0 = 2 * j
            pltpu.make_async_copy(hw2_hbm.at[sseg.at[b0]], rows0, sem0).wait()
            pltpu.sync_copy(rows0, acc_s.at[dseg.at[b0]], add=True)

            @pl.when(b0 + 2 < SEG)
            def _():
                _fire(b0 + 2, rows0, sem0)

            b1 = b0 + 1
            pltpu.make_async_copy(hw2_hbm.at[sseg.at[b1]], rows1, sem1).wait()
            pltpu.sync_copy(rows1, acc_s.at[dseg.at[b1]], add=True)

            @pl.when(b1 + 2 < SEG)
            def _():
                _fire(b1 + 2, rows1, sem1)

            return cc

        lax.fori_loop(0, SEG // 2, _pair, 0)
        pltpu.make_async_copy(hw2_hbm.at[sseg.at[SEG - 1]], rows0, sem0).wait()
        pltpu.sync_copy(rows0, acc_s.at[dseg.at[SEG - 1]], add=True)
        return carry

    lax.fori_loop(0, NSEG, _seg_body, 0)

    plsc.subcore_barrier()
    for k in range(SR // CH_E):
        pltpu.sync_copy(acc_s.at[pl.ds(tbase + k * CH_E, CH_E)], rows0)
        pltpu.sync_copy(rows0, out_hbm.at[pl.ds(cNP + tbase + k * CH_E, CH_E)])


@functools.partial(
    pl.kernel,
    out_type=(jax.ShapeDtypeStruct((2 * NP, DH), jnp.float32),
              jax.ShapeDtypeStruct((NC, NS, NB, 2, CH_E), jnp.float32),
              jax.ShapeDtypeStruct((4 * NP,), jnp.float32)),
    mesh=_mesh,
    scratch_types=[
        pltpu.VMEM((2, 6, CH_E), jnp.int32),
        pltpu.VMEM((2, 4, CH_E), jnp.float32),
        pltpu.VMEM((2, 2, CH_E), jnp.float32),
        pltpu.VMEM((SR,), jnp.float32),
        pltpu.VMEM((SR,), jnp.float32),
        pltpu.VMEM((SR,), jnp.float32),
        pltpu.VMEM((SR,), jnp.float32),
        pltpu.VMEM((2, 128), jnp.float32),
        pltpu.VMEM((CH_E, DH), jnp.float32),
        pltpu.VMEM((CH_E, DH), jnp.float32),
        pltpu.VMEM((CH_E, DH), jnp.float32),
        pltpu.SemaphoreType.DMA,
        pltpu.SemaphoreType.DMA,
        pltpu.VMEM_SHARED((NP,), jnp.float32),
        pltpu.VMEM_SHARED((NP,), jnp.float32),
        pltpu.VMEM_SHARED((NP, DH), jnp.float32),
    ],
)
def _sc_gat(xl2_hbm, al_hbm, ar_hbm, m2_hbm, src3_hbm, dst3_hbm,
            out_hbm, ee_hbm, dn_hbm,
            islot, fslot, eslot, stA, stB, esA, esB, mv,
            rows0, rows1, obuf, sem0, sem1,
            dn_s0, dn_s1, acc_s):
    c = lax.axis_index("c")
    t = lax.axis_index("s")
    cNP = c * NP
    c2NP = c * (2 * NP)
    tbase = t * SR
    rowbufs = (rows0, rows1)
    sems = (sem0, sem1)

    pltpu.sync_copy(m2_hbm, mv)
    zmax = mv[0, pl.ds(0, L)] + mv[1, pl.ds(0, L)]
    m = jnp.where(zmax >= 0.0, zmax, 0.2 * zmax)

    def _leaky_ee(a, b):
        z = a + b
        z = jnp.where(z >= 0.0, z, 0.2 * z)
        return jnp.exp(z - m)

    for h, es in ((0, esA), (1, esB)):
        off = c2NP + h * NP + tbase
        pltpu.sync_copy(al_hbm.at[pl.ds(off, SR)], es)
        pltpu.sync_copy(ar_hbm.at[pl.ds(off, SR)], stA)
        for j in range(SR // L):
            sl = pl.ds(j * L, L)
            es[sl] = _leaky_ee(es[sl], stA[sl])
    pltpu.sync_copy(esA, dn_s0.at[pl.ds(tbase, SR)])
    pltpu.sync_copy(esB, dn_s1.at[pl.ds(tbase, SR)])
    plsc.subcore_barrier()

    def _load_idx(b, s, xl_off):
        pltpu.sync_copy(src3_hbm.at[t, b], islot.at[s, 0])
        pltpu.sync_copy(dst3_hbm.at[t, b], islot.at[s, 1])
        for j in range(CH_E // L):
            sl = pl.ds(j * L, L)
            sv = islot[s, 0, sl]
            dv = islot[s, 1, sl]
            islot[s, 2, sl] = sv + xl_off
            islot[s, 3, sl] = sv + (c2NP + NP)
            islot[s, 4, sl] = dv + c2NP
            islot[s, 5, sl] = dv + (c2NP + NP)

    def _fire_att(b, s):
        _load_idx(b, s, c2NP)
        pltpu.async_copy(al_hbm.at[islot.at[s, 2]], fslot.at[s, 0], sems[s])
        pltpu.async_copy(al_hbm.at[islot.at[s, 3]], fslot.at[s, 1], sems[s])
        pltpu.async_copy(ar_hbm.at[islot.at[s, 4]], fslot.at[s, 2], sems[s])
        pltpu.async_copy(ar_hbm.at[islot.at[s, 5]], fslot.at[s, 3], sems[s])

    def _proc_att(b, s):
        for k in range(4):
            pltpu.make_async_copy(al_hbm.at[islot.at[s, 2]],
                                  fslot.at[s, k], sems[s]).wait()
        for h in range(2):
            for j in range(CH_E // L):
                sl = pl.ds(j * L, L)
                fslot[s, h, sl] = _leaky_ee(fslot[s, h, sl],
                                            fslot[s, h + 2, sl])
        pltpu.sync_copy(fslot.at[s, 0], dn_s0.at[islot.at[s, 1]], add=True)
        pltpu.sync_copy(fslot.at[s, 1], dn_s1.at[islot.at[s, 1]], add=True)
        pltpu.sync_copy(fslot.at[s, 0], ee_hbm.at[c, t, b, 0])
        pltpu.sync_copy(fslot.at[s, 1], ee_hbm.at[c, t, b, 1])

    _fire_att(0, 0)

    def _pairA(j, carry):
        b0 = 2 * j
        _fire_att(b0 + 1, 1)
        _proc_att(b0, 0)
        _fire_att(b0 + 2, 0)
        _proc_att(b0 + 1, 1)
        return carry

    lax.fori_loop(0, NB // 2, _pairA, 0)
    _proc_att(NB - 1, 0)
    plsc.subcore_barrier()

    pltpu.sync_copy(dn_s0.at[pl.ds(tbase, SR)], stA)
    pltpu.sync_copy(dn_s1.at[pl.ds(tbase, SR)], stB)
    pltpu.sync_copy(stA, dn_hbm.at[pl.ds(c2NP + tbase, SR)])
    pltpu.sync_copy(stB, dn_hbm.at[pl.ds(c2NP + NP + tbase, SR)])

    def _scale_chunk(a0v, a1v, j, rowbuf):
        for rl in range(L):
            r = j * L + rl
            s0 = jnp.full((L,), a0v[rl], jnp.float32)
            s1 = jnp.full((L,), a1v[rl], jnp.float32)
            for q in range(64 // L):
                sl = pl.ds(q * L, L)
                obuf[r, sl] = rowbuf[r, sl] * s0
            for q in range(64 // L):
                sl = pl.ds(64 + q * L, L)
                obuf[r, sl] = rowbuf[r, sl] * s1

    for k in range(SR // CH_E):
        pltpu.sync_copy(xl2_hbm.at[pl.ds(cNP + tbase + k * CH_E, CH_E)], rows0)

        def _self_j(j, carry, k=k):
            sl = pl.ds(k * CH_E + j * L, L)
            a0v = esA[sl] / (stA[sl] + 1e-16)
            a1v = esB[sl] / (stB[sl] + 1e-16)
            _scale_chunk(a0v, a1v, j, rows0)
            return carry

        lax.fori_loop(0, CH_E // L, _self_j, 0)
        pltpu.sync_copy(obuf, acc_s.at[pl.ds(tbase + k * CH_E, CH_E)])
    plsc.subcore_barrier()

    def _fire_msg(b, s):
        _load_idx(b, s, cNP)
        pltpu.async_copy(ee_hbm.at[c, t, b], eslot.at[s], sems[s])
        pltpu.async_copy(dn_hbm.at[islot.at[s, 4]], fslot.at[s, 2], sems[s])
        pltpu.async_copy(dn_hbm.at[islot.at[s, 5]], fslot.at[s, 3], sems[s])
        pltpu.async_copy(xl2_hbm.at[islot.at[s, 2]], rowbufs[s], sems[s])

    def _proc_msg(b, s):
        pltpu.make_async_copy(ee_hbm.at[c, t, b], eslot.at[s], sems[s]).wait()
        for k in range(2, 4):
            pltpu.make_async_copy(dn_hbm.at[islot.at[s, 4]],
                                  fslot.at[s, k], sems[s]).wait()
        pltpu.make_async_copy(xl2_hbm.at[islot.at[s, 2]],
                              rowbufs[s], sems[s]).wait()
        def _msg_j(j, carry):
            sl = pl.ds(j * L, L)
            a0v = eslot[s, 0, sl] / (fslot[s, 2, sl] + 1e-16)
            a1v = eslot[s, 1, sl] / (fslot[s, 3, sl] + 1e-16)
            _scale_chunk(a0v, a1v, j, rowbufs[s])
            return carry

        lax.fori_loop(0, CH_E // L, _msg_j, 0)
        pltpu.sync_copy(obuf, acc_s.at[islot.at[s, 1]], add=True)

    _fire_msg(0, 0)

    def _pairB(j, carry):
        b0 = 2 * j
        _fire_msg(b0 + 1, 1)
        _proc_msg(b0, 0)
        _fire_msg(b0 + 2, 0)
        _proc_msg(b0 + 1, 1)
        return carry

    lax.fori_loop(0, NB // 2, _pairB, 0)
    _proc_msg(NB - 1, 0)
    plsc.subcore_barrier()

    for k in range(SR // CH_E):
        pltpu.sync_copy(acc_s.at[pl.ds(tbase + k * CH_E, CH_E)], rows0)
        pltpu.sync_copy(rows0, out_hbm.at[pl.ds(cNP + tbase + k * CH_E, CH_E)])


BR = 640
GRID = NP // BR


def _dot(a, b):
    return jnp.dot(a, b, preferred_element_type=jnp.float32)


def _row_spec(cols):
    return pl.BlockSpec((BR, cols), lambda i: (i, 0))


def _half_spec():
    return pl.BlockSpec((2, BR, DH), lambda i: (0, i, 0))


def _full_spec(shape):
    nd = len(shape)
    return pl.BlockSpec(shape, lambda i: (0,) * nd)


def _tc_mm1_body(x_ref, wi_ref, bi_ref, wg_ref, deg_ref, h0_ref, hw_ref):
    h0 = jnp.maximum(_dot(x_ref[...], wi_ref[...]) + bi_ref[...], 0.0)
    dinv = lax.rsqrt(deg_ref[...])
    z = _dot(h0, wg_ref[...]) * dinv
    h0_ref[...] = h0
    hw_ref[0] = z[:, :DH]
    hw_ref[1] = z[:, DH:]


def _tc_mm1(x_p, W_in, b_in, W_g1, deg):
    return pl.pallas_call(
        _tc_mm1_body,
        grid=(GRID,),
        in_specs=[_row_spec(D), _full_spec((D, D)), _full_spec((1, D)),
                  _full_spec((D, D)), _row_spec(1)],
        out_specs=(_row_spec(D), _half_spec()),
        out_shape=(jax.ShapeDtypeStruct((NP, D), jnp.float32),
                   jax.ShapeDtypeStruct((2, NP, DH), jnp.float32)),
    )(x_p, W_in, b_in, W_g1, deg)


def _tc_stat_body(acc_ref, deg_ref, b_ref, o_ref):
    i = pl.program_id(0)
    y = jnp.concatenate([acc_ref[0], acc_ref[1]], axis=1)
    y = y * lax.rsqrt(deg_ref[...]) + b_ref[...]
    rid = lax.broadcasted_iota(jnp.int32, (BR, 1), 0) + i * BR
    mask = rid < N
    y = jnp.where(mask, y, 0.0)
    s1 = jnp.sum(y, axis=0, keepdims=True)
    s2 = jnp.sum(y * y, axis=0, keepdims=True)
    blk = jnp.concatenate([s1, s2], axis=0)

    @pl.when(i == 0)
    def _():
        o_ref[...] = blk

    @pl.when(i > 0)
    def _():
        o_ref[...] = o_ref[...] + blk


def _tc_stat(acc, deg, b):
    return pl.pallas_call(
        _tc_stat_body,
        grid=(GRID,),
        in_specs=[_half_spec(), _row_spec(1), _full_spec((1, D))],
        out_specs=_full_spec((2, D)),
        out_shape=jax.ShapeDtypeStruct((2, D), jnp.float32),
    )(acc, deg, b)


def _bn_from_stats(y, st_ref, gamma_ref, beta_ref):
    mu = st_ref[0:1, :] / FN
    var = st_ref[1:2, :] / FN - mu * mu
    return (y - mu) * lax.rsqrt(var + 1e-5) * gamma_ref[...] + beta_ref[...]


def _tc_fuse1_body(h_ref, acc_ref, deg_ref, b_ref, g_ref, be_ref, st_ref,
                   w_ref, h1_ref, hw_ref):
    dinv = lax.rsqrt(deg_ref[...])
    y = jnp.concatenate([acc_ref[0], acc_ref[1]], axis=1) * dinv + b_ref[...]
    h1 = h_ref[...] + jnp.maximum(_bn_from_stats(y, st_ref, g_ref, be_ref), 0.0)
    z = _dot(h1, w_ref[...]) * dinv
    h1_ref[...] = h1
    hw_ref[0] = z[:, :DH]
    hw_ref[1] = z[:, DH:]


def _tc_fuse1(h, acc, deg, b, gamma, beta, st, W_next):
    return pl.pallas_call(
        _tc_fuse1_body,
        grid=(GRID,),
        in_specs=[_row_spec(D), _half_spec(), _row_spec(1),
                  _full_spec((1, D)), _full_spec((1, D)), _full_spec((1, D)),
                  _full_spec((2, D)), _full_spec((D, D))],
        out_specs=(_row_spec(D), _half_spec()),
        out_shape=(jax.ShapeDtypeStruct((NP, D), jnp.float32),
                   jax.ShapeDtypeStruct((2, NP, DH), jnp.float32)),
    )(h, acc, deg, b, gamma, beta, st, W_next)


def _tc_fuse2_body(h_ref, acc_ref, deg_ref, b_ref, g_ref, be_ref, st_ref,
                   wg_ref, asrc_ref, adst_ref,
                   h2_ref, xl_ref, al_ref, ar_ref, m_ref):
    i = pl.program_id(0)
    dinv = lax.rsqrt(deg_ref[...])
    y = jnp.concatenate([acc_ref[0], acc_ref[1]], axis=1) * dinv + b_ref[...]
    h2 = h_ref[...] + jnp.maximum(_bn_from_stats(y, st_ref, g_ref, be_ref), 0.0)
    xl = _dot(h2, wg_ref[...])
    asrc = asrc_ref[...]
    adst = adst_ref[...]
    al_cols = []
    ar_cols = []
    for h in range(HEADS):
        xh = xl[:, 64 * h:64 * (h + 1)]
        al_cols.append(jnp.sum(xh * asrc[h:h + 1, :], axis=1, keepdims=True))
        ar_cols.append(jnp.sum(xh * adst[h:h + 1, :], axis=1, keepdims=True))
    al = jnp.concatenate(al_cols, axis=1)
    ar = jnp.concatenate(ar_cols, axis=1)
    h2_ref[...] = h2
    xl_ref[0] = xl[:, :DH]
    xl_ref[1] = xl[:, DH:]
    for ci in range(2):
        for h in range(2):
            al_ref[ci, h, :] = al[:, 2 * ci + h]
            ar_ref[ci, h, :] = ar[:, 2 * ci + h]
    bal = jnp.full((1, 128), jnp.max(al), jnp.float32)
    bar = jnp.full((1, 128), jnp.max(ar), jnp.float32)
    blk = jnp.concatenate([bal, bar], axis=0)

    @pl.when(i == 0)
    def _():
        m_ref[...] = blk

    @pl.when(i > 0)
    def _():
        m_ref[...] = jnp.maximum(m_ref[...], blk)


def _tc_fuse2(h, acc, deg, b, gamma, beta, st, W_gat, att_src, att_dst):
    return pl.pallas_call(
        _tc_fuse2_body,
        grid=(GRID,),
        in_specs=[_row_spec(D), _half_spec(), _row_spec(1),
                  _full_spec((1, D)), _full_spec((1, D)), _full_spec((1, D)),
                  _full_spec((2, D)), _full_spec((D, D)),
                  _full_spec((HEADS, 64)), _full_spec((HEADS, 64))],
        out_specs=(_row_spec(D), _half_spec(),
                   pl.BlockSpec((2, 2, BR), lambda i: (0, 0, i)),
                   pl.BlockSpec((2, 2, BR), lambda i: (0, 0, i)),
                   _full_spec((2, 128))),
        out_shape=(jax.ShapeDtypeStruct((NP, D), jnp.float32),
                   jax.ShapeDtypeStruct((2, NP, DH), jnp.float32),
                   jax.ShapeDtypeStruct((2, 2, NP), jnp.float32),
                   jax.ShapeDtypeStruct((2, 2, NP), jnp.float32),
                   jax.ShapeDtypeStruct((2, 128), jnp.float32)),
    )(h, acc, deg, b, gamma, beta, st, W_gat, att_src, att_dst)


def _tc_head_body(hg_ref, h2_ref, b_ref, g_ref, be_ref, st_ref,
                  w1_ref, b1_ref, w2_ref, b2_ref, o_ref):
    y = jnp.concatenate([hg_ref[0], hg_ref[1]], axis=1) + b_ref[...]
    h3 = _bn_from_stats(y, st_ref, g_ref, be_ref) + h2_ref[...]
    tmid = jnp.maximum(_dot(h3, w1_ref[...]) + b1_ref[...], 0.0)
    o_ref[...] = _dot(tmid, w2_ref[...]) + b2_ref[...]


def _tc_head(hg, h2, b_gat, gamma_g, beta_g, st, W_h1, b_h1, W_h2, b_h2):
    return pl.pallas_call(
        _tc_head_body,
        grid=(GRID,),
        in_specs=[_half_spec(), _row_spec(D), _full_spec((1, D)),
                  _full_spec((1, D)), _full_spec((1, D)), _full_spec((2, D)),
                  _full_spec((D, 128)), _full_spec((1, 128)),
                  _full_spec((128, 1)), _full_spec((1, 1))],
        out_specs=_row_spec(1),
        out_shape=jax.ShapeDtypeStruct((NP, 1), jnp.float32),
    )(hg, h2, b_gat, gamma_g, beta_g, st, W_h1, b_h1, W_h2, b_h2)


_DBG_DEG = False
_DBG_GCN = False
_DBG_GAT = False


def _jnp_deg(dst_t):
    d = dst_t.reshape(-1)
    deg = jax.ops.segment_sum(jnp.ones((E,), jnp.float32), d, num_segments=NP)
    return deg + 1.0


def _jnp_gcn(hw2f, src_t, dst_t):
    hw = hw2f.reshape(2, NP, DH)
    s = src_t.reshape(-1)
    d = dst_t.reshape(-1)
    agg = hw + jax.vmap(
        lambda h: jax.ops.segment_sum(h[s], d, num_segments=NP))(hw)
    return agg.reshape(2 * NP, DH)


def _jnp_gat(xl2f, al_f, ar_f, m2, src_t, dst_t):
    xl = xl2f.reshape(2, NP, DH)
    al = al_f.reshape(2, 2, NP)
    ar = ar_f.reshape(2, 2, NP)
    s = src_t.reshape(-1)
    d = dst_t.reshape(-1)
    zm = m2[0, 0] + m2[1, 0]
    m = jnp.where(zm >= 0, zm, 0.2 * zm)
    outs = []
    for c in range(2):
        halves = []
        for h in range(2):
            z = al[c, h][s] + ar[c, h][d]
            z = jnp.where(z >= 0, z, 0.2 * z)
            ee = jnp.exp(z - m)
            zs = al[c, h] + ar[c, h]
            zs = jnp.where(zs >= 0, zs, 0.2 * zs)
            ees = jnp.exp(zs - m)
            dn = jax.ops.segment_sum(ee, d, num_segments=NP) + ees
            xh = xl[c][:, 64 * h:64 * (h + 1)]
            msg = jax.ops.segment_sum(xh[s] * (ee / (dn[d] + 1e-16))[:, None],
                                      d, num_segments=NP)
            msg = msg + xh * (ees / (dn + 1e-16))[:, None]
            halves.append(msg)
        outs.append(jnp.concatenate(halves, axis=1))
    return jnp.stack(outs, 0).reshape(2 * NP, DH)


@jax.jit
def kernel(x, edge_index, W_in, b_in, W_g1, b_g1, gamma1, beta1,
           W_g2, b_g2, gamma2, beta2, W_gat, att_src, att_dst, b_gat,
           gamma_g, beta_g, W_h1, b_h1, W_h2, b_h2):
    x_p = jnp.pad(x, ((0, NP - N), (0, 0)))
    src_t = edge_index[0].reshape(NS, NSEG, SEG, CH_E)
    dst_t = edge_index[1].reshape(NS, NSEG, SEG, CH_E)
    src_e3 = edge_index[0].reshape(NS, NB, CH_E)
    dst_e3 = edge_index[1].reshape(NS, NB, CH_E)
    ones_sr = jnp.ones((SR,), jnp.float32)

    r2 = lambda v: v[None, :]
    b_in2, b_g1_2, b_g2_2, b_gat2 = r2(b_in), r2(b_g1), r2(b_g2), r2(b_gat)
    gamma1_2, beta1_2 = r2(gamma1), r2(beta1)
    gamma2_2, beta2_2 = r2(gamma2), r2(beta2)
    gamma_g2, beta_g2 = r2(gamma_g), r2(beta_g)
    b_h1_2 = r2(b_h1)
    b_h2_2 = r2(b_h2)

    deg = _jnp_deg(dst_t) if _DBG_DEG else _sc_deg(dst_t, ones_sr)
    deg2 = deg[:, None]

    h0, hw1 = _tc_mm1(x_p, W_in, b_in2, W_g1, deg2)
    agg1 = (_jnp_gcn(hw1.reshape(2 * NP, DH), src_t, dst_t) if _DBG_GCN
            else _sc_gcn(hw1.reshape(2 * NP, DH), src_t, dst_t)).reshape(2, NP, DH)
    st1 = _tc_stat(agg1, deg2, b_g1_2)
    h1, hw2 = _tc_fuse1(h0, agg1, deg2, b_g1_2, gamma1_2, beta1_2, st1, W_g2)
    agg2 = (_jnp_gcn(hw2.reshape(2 * NP, DH), src_t, dst_t) if _DBG_GCN
            else _sc_gcn(hw2.reshape(2 * NP, DH), src_t, dst_t)).reshape(2, NP, DH)
    st2 = _tc_stat(agg2, deg2, b_g2_2)
    h2, xl2, al_ph, ar_ph, m2 = _tc_fuse2(
        h1, agg2, deg2, b_g2_2, gamma2_2, beta2_2, st2,
        W_gat, att_src, att_dst)
    if _DBG_GAT:
        hg = _jnp_gat(xl2.reshape(2 * NP, DH), al_ph.reshape(4 * NP),
                      ar_ph.reshape(4 * NP), m2, src_t, dst_t).reshape(2, NP, DH)
    else:
        hg = _sc_gat(xl2.reshape(2 * NP, DH), al_ph.reshape(4 * NP),
                     ar_ph.reshape(4 * NP), m2, src_e3,
                     dst_e3)[0].reshape(2, NP, DH)
    ones_np = jnp.ones((NP, 1), jnp.float32)
    st3 = _tc_stat(hg, ones_np, b_gat2)
    out = _tc_head(hg, h2, b_gat2, gamma_g2, beta_g2, st3,
                   W_h1, b_h1_2, W_h2, b_h2_2)
    return out[:N]


# speedup vs baseline: 33.8410x; 1.1264x over previous
import functools

import jax
import jax.numpy as jnp
from jax import lax
from jax.experimental import pallas as pl
from jax.experimental.pallas import tpu as pltpu
from jax.experimental.pallas import tpu_sc as plsc

N = 10000
NP = 10240
D = 256
DH = 128
E = 160000
HEADS = 4

NC = 2
NS = 16
L = 16
SR = NP // NS
EPT = E // NS
CH_E = 80
NB = EPT // CH_E
FN = float(N)
SEG = 25
NSEG = NB // SEG

_mesh = plsc.VectorSubcoreMesh(core_axis_name="c", subcore_axis_name="s",
                               num_cores=NC, num_subcores=NS)


@functools.partial(
    pl.kernel,
    out_type=jax.ShapeDtypeStruct((NP,), jnp.float32),
    mesh=_mesh,
    scratch_types=[
        pltpu.VMEM((SEG, CH_E), jnp.int32),
        pltpu.VMEM((SR,), jnp.float32),
        pltpu.VMEM_SHARED((NP,), jnp.float32),
    ],
)
def _sc_deg(dstT_hbm, ones_hbm, deg_hbm, dst_slab, onesv, deg_s):
    c = lax.axis_index("c")
    t = lax.axis_index("s")
    tbase = t * SR
    pltpu.sync_copy(ones_hbm, onesv)
    pltpu.sync_copy(onesv, deg_s.at[pl.ds(tbase, SR)])
    plsc.subcore_barrier()

    def _seg(seg, carry):
        pltpu.sync_copy(dstT_hbm.at[t, seg], dst_slab)

        def _b(b, cc):
            pltpu.sync_copy(onesv.at[pl.ds(0, CH_E)],
                            deg_s.at[dst_slab.at[b]], add=True)
            return cc

        lax.fori_loop(0, SEG, _b, 0)
        return carry

    lax.fori_loop(0, NSEG, _seg, 0)
    plsc.subcore_barrier()

    @pl.when(c == 0)
    def _():
        pltpu.sync_copy(deg_s.at[pl.ds(tbase, SR)], onesv)
        pltpu.sync_copy(onesv, deg_hbm.at[pl.ds(tbase, SR)])


@functools.partial(
    pl.kernel,
    out_type=jax.ShapeDtypeStruct((2 * NP, DH), jnp.float32),
    mesh=_mesh,
    scratch_types=[
        pltpu.VMEM((SEG, CH_E), jnp.int32),
        pltpu.VMEM((SEG, CH_E), jnp.int32),
        pltpu.VMEM((CH_E, DH), jnp.float32),
        pltpu.VMEM((CH_E, DH), jnp.float32),
        pltpu.SemaphoreType.DMA,
        pltpu.SemaphoreType.DMA,
        pltpu.VMEM_SHARED((NP, DH), jnp.float32),
    ],
)
def _sc_gcn(hw2_hbm, srcT_hbm, dstT_hbm, out_hbm,
            sseg, dseg, rows0, rows1, sem0, sem1, acc_s):
    c = lax.axis_index("c")
    t = lax.axis_index("s")
    cNP = c * NP
    tbase = t * SR

    for k in range(SR // CH_E):
        pltpu.sync_copy(hw2_hbm.at[pl.ds(cNP + tbase + k * CH_E, CH_E)], rows0)
        pltpu.sync_copy(rows0, acc_s.at[pl.ds(tbase + k * CH_E, CH_E)])
    plsc.subcore_barrier()

    def _fire(b, buf, sem):
        return pltpu.async_copy(hw2_hbm.at[sseg.at[b]], buf, sem)

    def _seg_body(seg, carry):
        pltpu.sync_copy(srcT_hbm.at[t, seg], sseg)
        pltpu.sync_copy(dstT_hbm.at[t, seg], dseg)

        def _adj(b, cc):
            for j in range(CH_E // L):
                sl = pl.ds(j * L, L)
                sseg[b, sl] = sseg[b, sl] + cNP
            return cc

        lax.fori_loop(0, SEG, _adj, 0)

        _fire(0, rows0, sem0)
        _fire(1, rows1, sem1)

        def _pair(j, cc):
            b0 = 2 * j
            pltpu.make_async_copy(hw2_hbm.at[sseg.at[b0]], rows0, sem0).wait()
            pltpu.sync_copy(rows0, acc_s.at[dseg.at[b0]], add=True)

            @pl.when(b0 + 2 < SEG)
            def _():
                _fire(b0 + 2, rows0, sem0)

            b1 = b0 + 1
            pltpu.make_async_copy(hw2_hbm.at[sseg.at[b1]], rows1, sem1).wait()
            pltpu.sync_copy(rows1, acc_s.at[dseg.at[b1]], add=True)

            @pl.when(b1 + 2 < SEG)
            def _():
                _fire(b1 + 2, rows1, sem1)

            return cc

        lax.fori_loop(0, SEG // 2, _pair, 0)
        pltpu.make_async_copy(hw2_hbm.at[sseg.at[SEG - 1]], rows0, sem0).wait()
        pltpu.sync_copy(rows0, acc_s.at[dseg.at[SEG - 1]], add=True)
        return carry

    lax.fori_loop(0, NSEG, _seg_body, 0)

    plsc.subcore_barrier()
    for k in range(SR // CH_E):
        pltpu.sync_copy(acc_s.at[pl.ds(tbase + k * CH_E, CH_E)], rows0)
        pltpu.sync_copy(rows0, out_hbm.at[pl.ds(cNP + tbase + k * CH_E, CH_E)])


@functools.partial(
    pl.kernel,
    out_type=(jax.ShapeDtypeStruct((2 * NP, DH), jnp.float32),
              jax.ShapeDtypeStruct((NC, NS, NB, 2, CH_E), jnp.float32)),
    mesh=_mesh,
    scratch_types=[
        pltpu.VMEM((2, 6, CH_E), jnp.int32),
        pltpu.VMEM((2, 4, CH_E), jnp.float32),
        pltpu.VMEM((2, 2, CH_E), jnp.float32),
        pltpu.VMEM((SR,), jnp.float32),
        pltpu.VMEM((SR,), jnp.float32),
        pltpu.VMEM((SR,), jnp.float32),
        pltpu.VMEM((SR,), jnp.float32),
        pltpu.VMEM((2, 128), jnp.float32),
        pltpu.VMEM((CH_E, DH), jnp.float32),
        pltpu.VMEM((CH_E, DH), jnp.float32),
        pltpu.SemaphoreType.DMA,
        pltpu.SemaphoreType.DMA,
        pltpu.VMEM_SHARED((NP,), jnp.float32),
        pltpu.VMEM_SHARED((NP,), jnp.float32),
        pltpu.VMEM_SHARED((NP, DH), jnp.float32),
    ],
)
def _sc_gat(xl2_hbm, al_hbm, ar_hbm, m2_hbm, sd3_hbm,
            out_hbm, ee_hbm,
            islot, fslot, eslot, stA, stB, esA, esB, mv,
            rows0, rows1, sem0, sem1,
            dn_s0, dn_s1, acc_s):
    c = lax.axis_index("c")
    t = lax.axis_index("s")
    cNP = c * NP
    c2NP = c * (2 * NP)
    tbase = t * SR
    rowbufs = (rows0, rows1)
    sems = (sem0, sem1)

    pltpu.sync_copy(m2_hbm, mv)
    zmax = mv[0, pl.ds(0, L)] + mv[1, pl.ds(0, L)]
    m = jnp.where(zmax >= 0.0, zmax, 0.2 * zmax)

    def _leaky_ee(a, b):
        z = a + b
        z = jnp.where(z >= 0.0, z, 0.2 * z)
        return jnp.exp(z - m)

    for h, es in ((0, esA), (1, esB)):
        off = c2NP + h * NP + tbase
        pltpu.sync_copy(al_hbm.at[pl.ds(off, SR)], es)
        pltpu.sync_copy(ar_hbm.at[pl.ds(off, SR)], stA)
        for j in range(SR // L):
            sl = pl.ds(j * L, L)
            es[sl] = _leaky_ee(es[sl], stA[sl])
    pltpu.sync_copy(esA, dn_s0.at[pl.ds(tbase, SR)])
    pltpu.sync_copy(esB, dn_s1.at[pl.ds(tbase, SR)])
    plsc.subcore_barrier()

    def _load_idx(b, s, xl_off):
        pltpu.sync_copy(sd3_hbm.at[t, b], islot.at[s, pl.ds(0, 2)])
        for j in range(CH_E // L):
            sl = pl.ds(j * L, L)
            sv = islot[s, 0, sl]
            dv = islot[s, 1, sl]
            islot[s, 2, sl] = sv + xl_off
            islot[s, 3, sl] = sv + (c2NP + NP)
            islot[s, 4, sl] = dv + c2NP
            islot[s, 5, sl] = dv + (c2NP + NP)

    def _fire_att(b, s):
        _load_idx(b, s, c2NP)
        pltpu.async_copy(al_hbm.at[islot.at[s, 2]], fslot.at[s, 0], sems[s])
        pltpu.async_copy(al_hbm.at[islot.at[s, 3]], fslot.at[s, 1], sems[s])
        pltpu.async_copy(ar_hbm.at[islot.at[s, 4]], fslot.at[s, 2], sems[s])
        pltpu.async_copy(ar_hbm.at[islot.at[s, 5]], fslot.at[s, 3], sems[s])

    def _proc_att(b, s):
        for k in range(4):
            pltpu.make_async_copy(al_hbm.at[islot.at[s, 2]],
                                  fslot.at[s, k], sems[s]).wait()
        for h in range(2):
            for j in range(CH_E // L):
                sl = pl.ds(j * L, L)
                fslot[s, h, sl] = _leaky_ee(fslot[s, h, sl],
                                            fslot[s, h + 2, sl])
        pltpu.sync_copy(fslot.at[s, 0], dn_s0.at[islot.at[s, 1]], add=True)
        pltpu.sync_copy(fslot.at[s, 1], dn_s1.at[islot.at[s, 1]], add=True)
        pltpu.sync_copy(fslot.at[s, pl.ds(0, 2)], ee_hbm.at[c, t, b])

    _fire_att(0, 0)

    def _pairA(j, carry):
        b0 = 2 * j
        _fire_att(b0 + 1, 1)
        _proc_att(b0, 0)
        _fire_att(b0 + 2, 0)
        _proc_att(b0 + 1, 1)
        return carry

    lax.fori_loop(0, NB // 2, _pairA, 0)
    _proc_att(NB - 1, 0)
    plsc.subcore_barrier()

    def _scale_chunk(a0v, a1v, j, rowbuf):
        for rl in range(L):
            r = j * L + rl
            s0 = jnp.full((L,), a0v[rl], jnp.float32)
            s1 = jnp.full((L,), a1v[rl], jnp.float32)
            for q in range(64 // L):
                sl = pl.ds(q * L, L)
                rowbuf[r, sl] = rowbuf[r, sl] * s0
            for q in range(64 // L):
                sl = pl.ds(64 + q * L, L)
                rowbuf[r, sl] = rowbuf[r, sl] * s1

    for k in range(SR // CH_E):
        pltpu.sync_copy(xl2_hbm.at[pl.ds(cNP + tbase + k * CH_E, CH_E)], rows0)

        def _self_j(j, carry, k=k):
            sl = pl.ds(k * CH_E + j * L, L)
            _scale_chunk(esA[sl], esB[sl], j, rows0)
            return carry

        lax.fori_loop(0, CH_E // L, _self_j, 0)
        pltpu.sync_copy(rows0, acc_s.at[pl.ds(tbase + k * CH_E, CH_E)])
    plsc.subcore_barrier()

    def _fire_msg(b, s):
        _load_idx(b, s, cNP)
        pltpu.async_copy(ee_hbm.at[c, t, b], eslot.at[s], sems[s])
        pltpu.async_copy(xl2_hbm.at[islot.at[s, 2]], rowbufs[s], sems[s])

    def _proc_msg(b, s):
        pltpu.make_async_copy(ee_hbm.at[c, t, b], eslot.at[s], sems[s]).wait()
        pltpu.make_async_copy(xl2_hbm.at[islot.at[s, 2]],
                              rowbufs[s], sems[s]).wait()

        def _msg_j(j, carry):
            sl = pl.ds(j * L, L)
            _scale_chunk(eslot[s, 0, sl], eslot[s, 1, sl], j, rowbufs[s])
            return carry

        lax.fori_loop(0, CH_E // L, _msg_j, 0)
        pltpu.sync_copy(rowbufs[s], acc_s.at[islot.at[s, 1]], add=True)

    _fire_msg(0, 0)

    def _pairB(j, carry):
        b0 = 2 * j
        _fire_msg(b0 + 1, 1)
        _proc_msg(b0, 0)
        _fire_msg(b0 + 2, 0)
        _proc_msg(b0 + 1, 1)
        return carry

    lax.fori_loop(0, NB // 2, _pairB, 0)
    _proc_msg(NB - 1, 0)
    plsc.subcore_barrier()

    pltpu.sync_copy(dn_s0.at[pl.ds(tbase, SR)], stA)
    pltpu.sync_copy(dn_s1.at[pl.ds(tbase, SR)], stB)
    for j in range(SR // L):
        sl = pl.ds(j * L, L)
        stA[sl] = 1.0 / (stA[sl] + 1e-16)
        stB[sl] = 1.0 / (stB[sl] + 1e-16)
    for k in range(SR // CH_E):
        pltpu.sync_copy(acc_s.at[pl.ds(tbase + k * CH_E, CH_E)], rows0)

        def _wb_j(j, carry, k=k):
            sl = pl.ds(k * CH_E + j * L, L)
            _scale_chunk(stA[sl], stB[sl], j, rows0)
            return carry

        lax.fori_loop(0, CH_E // L, _wb_j, 0)
        pltpu.sync_copy(rows0, out_hbm.at[pl.ds(cNP + tbase + k * CH_E, CH_E)])


BR = 640
GRID = NP // BR


def _dot(a, b):
    return jnp.dot(a, b, preferred_element_type=jnp.float32)


def _row_spec(cols):
    return pl.BlockSpec((BR, cols), lambda i: (i, 0))


def _half_spec():
    return pl.BlockSpec((2, BR, DH), lambda i: (0, i, 0))


def _full_spec(shape):
    nd = len(shape)
    return pl.BlockSpec(shape, lambda i: (0,) * nd)


def _tc_mm1_body(x_ref, wi_ref, bi_ref, wg_ref, deg_ref, h0_ref, hw_ref):
    h0 = jnp.maximum(_dot(x_ref[...], wi_ref[...]) + bi_ref[...], 0.0)
    dinv = lax.rsqrt(deg_ref[...])
    z = _dot(h0, wg_ref[...]) * dinv
    h0_ref[...] = h0
    hw_ref[0] = z[:, :DH]
    hw_ref[1] = z[:, DH:]


def _tc_mm1(x_p, W_in, b_in, W_g1, deg):
    return pl.pallas_call(
        _tc_mm1_body,
        grid=(GRID,),
        in_specs=[_row_spec(D), _full_spec((D, D)), _full_spec((1, D)),
                  _full_spec((D, D)), _row_spec(1)],
        out_specs=(_row_spec(D), _half_spec()),
        out_shape=(jax.ShapeDtypeStruct((NP, D), jnp.float32),
                   jax.ShapeDtypeStruct((2, NP, DH), jnp.float32)),
    )(x_p, W_in, b_in, W_g1, deg)


def _tc_stat_body(acc_ref, deg_ref, b_ref, o_ref):
    i = pl.program_id(0)
    y = jnp.concatenate([acc_ref[0], acc_ref[1]], axis=1)
    y = y * lax.rsqrt(deg_ref[...]) + b_ref[...]
    rid = lax.broadcasted_iota(jnp.int32, (BR, 1), 0) + i * BR
    mask = rid < N
    y = jnp.where(mask, y, 0.0)
    s1 = jnp.sum(y, axis=0, keepdims=True)
    s2 = jnp.sum(y * y, axis=0, keepdims=True)
    blk = jnp.concatenate([s1, s2], axis=0)

    @pl.when(i == 0)
    def _():
        o_ref[...] = blk

    @pl.when(i > 0)
    def _():
        o_ref[...] = o_ref[...] + blk


def _tc_stat(acc, deg, b):
    return pl.pallas_call(
        _tc_stat_body,
        grid=(GRID,),
        in_specs=[_half_spec(), _row_spec(1), _full_spec((1, D))],
        out_specs=_full_spec((2, D)),
        out_shape=jax.ShapeDtypeStruct((2, D), jnp.float32),
    )(acc, deg, b)


def _bn_from_stats(y, st_ref, gamma_ref, beta_ref):
    mu = st_ref[0:1, :] / FN
    var = st_ref[1:2, :] / FN - mu * mu
    return (y - mu) * lax.rsqrt(var + 1e-5) * gamma_ref[...] + beta_ref[...]


def _tc_fuse1_body(h_ref, acc_ref, deg_ref, b_ref, g_ref, be_ref, st_ref,
                   w_ref, h1_ref, hw_ref):
    dinv = lax.rsqrt(deg_ref[...])
    y = jnp.concatenate([acc_ref[0], acc_ref[1]], axis=1) * dinv + b_ref[...]
    h1 = h_ref[...] + jnp.maximum(_bn_from_stats(y, st_ref, g_ref, be_ref), 0.0)
    z = _dot(h1, w_ref[...]) * dinv
    h1_ref[...] = h1
    hw_ref[0] = z[:, :DH]
    hw_ref[1] = z[:, DH:]


def _tc_fuse1(h, acc, deg, b, gamma, beta, st, W_next):
    return pl.pallas_call(
        _tc_fuse1_body,
        grid=(GRID,),
        in_specs=[_row_spec(D), _half_spec(), _row_spec(1),
                  _full_spec((1, D)), _full_spec((1, D)), _full_spec((1, D)),
                  _full_spec((2, D)), _full_spec((D, D))],
        out_specs=(_row_spec(D), _half_spec()),
        out_shape=(jax.ShapeDtypeStruct((NP, D), jnp.float32),
                   jax.ShapeDtypeStruct((2, NP, DH), jnp.float32)),
    )(h, acc, deg, b, gamma, beta, st, W_next)


def _tc_fuse2_body(h_ref, acc_ref, deg_ref, b_ref, g_ref, be_ref, st_ref,
                   wg_ref, asrc_ref, adst_ref,
                   h2_ref, xl_ref, al_ref, ar_ref, m_ref):
    i = pl.program_id(0)
    dinv = lax.rsqrt(deg_ref[...])
    y = jnp.concatenate([acc_ref[0], acc_ref[1]], axis=1) * dinv + b_ref[...]
    h2 = h_ref[...] + jnp.maximum(_bn_from_stats(y, st_ref, g_ref, be_ref), 0.0)
    xl = _dot(h2, wg_ref[...])
    asrc = asrc_ref[...]
    adst = adst_ref[...]
    al_cols = []
    ar_cols = []
    for h in range(HEADS):
        xh = xl[:, 64 * h:64 * (h + 1)]
        al_cols.append(jnp.sum(xh * asrc[h:h + 1, :], axis=1, keepdims=True))
        ar_cols.append(jnp.sum(xh * adst[h:h + 1, :], axis=1, keepdims=True))
    al = jnp.concatenate(al_cols, axis=1)
    ar = jnp.concatenate(ar_cols, axis=1)
    h2_ref[...] = h2
    xl_ref[0] = xl[:, :DH]
    xl_ref[1] = xl[:, DH:]
    for ci in range(2):
        for h in range(2):
            al_ref[ci, h, :] = al[:, 2 * ci + h]
            ar_ref[ci, h, :] = ar[:, 2 * ci + h]
    bal = jnp.full((1, 128), jnp.max(al), jnp.float32)
    bar = jnp.full((1, 128), jnp.max(ar), jnp.float32)
    blk = jnp.concatenate([bal, bar], axis=0)

    @pl.when(i == 0)
    def _():
        m_ref[...] = blk

    @pl.when(i > 0)
    def _():
        m_ref[...] = jnp.maximum(m_ref[...], blk)


def _tc_fuse2(h, acc, deg, b, gamma, beta, st, W_gat, att_src, att_dst):
    return pl.pallas_call(
        _tc_fuse2_body,
        grid=(GRID,),
        in_specs=[_row_spec(D), _half_spec(), _row_spec(1),
                  _full_spec((1, D)), _full_spec((1, D)), _full_spec((1, D)),
                  _full_spec((2, D)), _full_spec((D, D)),
                  _full_spec((HEADS, 64)), _full_spec((HEADS, 64))],
        out_specs=(_row_spec(D), _half_spec(),
                   pl.BlockSpec((2, 2, BR), lambda i: (0, 0, i)),
                   pl.BlockSpec((2, 2, BR), lambda i: (0, 0, i)),
                   _full_spec((2, 128))),
        out_shape=(jax.ShapeDtypeStruct((NP, D), jnp.float32),
                   jax.ShapeDtypeStruct((2, NP, DH), jnp.float32),
                   jax.ShapeDtypeStruct((2, 2, NP), jnp.float32),
                   jax.ShapeDtypeStruct((2, 2, NP), jnp.float32),
                   jax.ShapeDtypeStruct((2, 128), jnp.float32)),
    )(h, acc, deg, b, gamma, beta, st, W_gat, att_src, att_dst)


def _tc_head_body(hg_ref, h2_ref, b_ref, g_ref, be_ref, st_ref,
                  w1_ref, b1_ref, w2_ref, b2_ref, o_ref):
    y = jnp.concatenate([hg_ref[0], hg_ref[1]], axis=1) + b_ref[...]
    h3 = _bn_from_stats(y, st_ref, g_ref, be_ref) + h2_ref[...]
    tmid = jnp.maximum(_dot(h3, w1_ref[...]) + b1_ref[...], 0.0)
    o_ref[...] = _dot(tmid, w2_ref[...]) + b2_ref[...]


def _tc_head(hg, h2, b_gat, gamma_g, beta_g, st, W_h1, b_h1, W_h2, b_h2):
    return pl.pallas_call(
        _tc_head_body,
        grid=(GRID,),
        in_specs=[_half_spec(), _row_spec(D), _full_spec((1, D)),
                  _full_spec((1, D)), _full_spec((1, D)), _full_spec((2, D)),
                  _full_spec((D, 128)), _full_spec((1, 128)),
                  _full_spec((128, 1)), _full_spec((1, 1))],
        out_specs=_row_spec(1),
        out_shape=jax.ShapeDtypeStruct((NP, 1), jnp.float32),
    )(hg, h2, b_gat, gamma_g, beta_g, st, W_h1, b_h1, W_h2, b_h2)


_DBG_DEG = False
_DBG_GCN = False
_DBG_GAT = False


def _jnp_deg(dst_t):
    d = dst_t.reshape(-1)
    deg = jax.ops.segment_sum(jnp.ones((E,), jnp.float32), d, num_segments=NP)
    return deg + 1.0


def _jnp_gcn(hw2f, src_t, dst_t):
    hw = hw2f.reshape(2, NP, DH)
    s = src_t.reshape(-1)
    d = dst_t.reshape(-1)
    agg = hw + jax.vmap(
        lambda h: jax.ops.segment_sum(h[s], d, num_segments=NP))(hw)
    return agg.reshape(2 * NP, DH)


def _jnp_gat(xl2f, al_f, ar_f, m2, src_t, dst_t):
    xl = xl2f.reshape(2, NP, DH)
    al = al_f.reshape(2, 2, NP)
    ar = ar_f.reshape(2, 2, NP)
    s = src_t.reshape(-1)
    d = dst_t.reshape(-1)
    zm = m2[0, 0] + m2[1, 0]
    m = jnp.where(zm >= 0, zm, 0.2 * zm)
    outs = []
    for c in range(2):
        halves = []
        for h in range(2):
            z = al[c, h][s] + ar[c, h][d]
            z = jnp.where(z >= 0, z, 0.2 * z)
            ee = jnp.exp(z - m)
            zs = al[c, h] + ar[c, h]
            zs = jnp.where(zs >= 0, zs, 0.2 * zs)
            ees = jnp.exp(zs - m)
            dn = jax.ops.segment_sum(ee, d, num_segments=NP) + ees
            xh = xl[c][:, 64 * h:64 * (h + 1)]
            msg = jax.ops.segment_sum(xh[s] * (ee / (dn[d] + 1e-16))[:, None],
                                      d, num_segments=NP)
            msg = msg + xh * (ees / (dn + 1e-16))[:, None]
            halves.append(msg)
        outs.append(jnp.concatenate(halves, axis=1))
    return jnp.stack(outs, 0).reshape(2 * NP, DH)


@jax.jit
def kernel(x, edge_index, W_in, b_in, W_g1, b_g1, gamma1, beta1,
           W_g2, b_g2, gamma2, beta2, W_gat, att_src, att_dst, b_gat,
           gamma_g, beta_g, W_h1, b_h1, W_h2, b_h2):
    x_p = jnp.pad(x, ((0, NP - N), (0, 0)))
    src_t = edge_index[0].reshape(NS, NSEG, SEG, CH_E)
    dst_t = edge_index[1].reshape(NS, NSEG, SEG, CH_E)
    sd3 = jnp.stack([edge_index[0].reshape(NS, NB, CH_E),
                     edge_index[1].reshape(NS, NB, CH_E)], axis=2)
    ones_sr = jnp.ones((SR,), jnp.float32)

    r2 = lambda v: v[None, :]
    b_in2, b_g1_2, b_g2_2, b_gat2 = r2(b_in), r2(b_g1), r2(b_g2), r2(b_gat)
    gamma1_2, beta1_2 = r2(gamma1), r2(beta1)
    gamma2_2, beta2_2 = r2(gamma2), r2(beta2)
    gamma_g2, beta_g2 = r2(gamma_g), r2(beta_g)
    b_h1_2 = r2(b_h1)
    b_h2_2 = r2(b_h2)

    deg = _jnp_deg(dst_t) if _DBG_DEG else _sc_deg(dst_t, ones_sr)
    deg2 = deg[:, None]

    h0, hw1 = _tc_mm1(x_p, W_in, b_in2, W_g1, deg2)
    agg1 = (_jnp_gcn(hw1.reshape(2 * NP, DH), src_t, dst_t) if _DBG_GCN
            else _sc_gcn(hw1.reshape(2 * NP, DH), src_t, dst_t)).reshape(2, NP, DH)
    st1 = _tc_stat(agg1, deg2, b_g1_2)
    h1, hw2 = _tc_fuse1(h0, agg1, deg2, b_g1_2, gamma1_2, beta1_2, st1, W_g2)
    agg2 = (_jnp_gcn(hw2.reshape(2 * NP, DH), src_t, dst_t) if _DBG_GCN
            else _sc_gcn(hw2.reshape(2 * NP, DH), src_t, dst_t)).reshape(2, NP, DH)
    st2 = _tc_stat(agg2, deg2, b_g2_2)
    h2, xl2, al_ph, ar_ph, m2 = _tc_fuse2(
        h1, agg2, deg2, b_g2_2, gamma2_2, beta2_2, st2,
        W_gat, att_src, att_dst)
    if _DBG_GAT:
        hg = _jnp_gat(xl2.reshape(2 * NP, DH), al_ph.reshape(4 * NP),
                      ar_ph.reshape(4 * NP), m2, src_t, dst_t).reshape(2, NP, DH)
    else:
        hg = _sc_gat(xl2.reshape(2 * NP, DH), al_ph.reshape(4 * NP),
                     ar_ph.reshape(4 * NP), m2, sd3)[0].reshape(2, NP, DH)
    ones_np = jnp.ones((NP, 1), jnp.float32)
    st3 = _tc_stat(hg, ones_np, b_gat2)
    out = _tc_head(hg, h2, b_gat2, gamma_g2, beta_g2, st3,
                   W_h1, b_h1_2, W_h2, b_h2_2)
    return out[:N]
